# baseline (speedup 1.0000x reference)
import functools
import math

import jax
import jax.numpy as jnp
import numpy as np
from jax import lax
from jax.experimental import pallas as pl
from jax.experimental.pallas import tpu as pltpu
from jax.experimental.pallas import tpu_sc as plsc

VOCAB = 100000
EMB_DIM = 1024
BATCH = 4
SEQ = 4096
SCALE = math.sqrt(float(EMB_DIM))

_NC = 2
_NS = 16
_NW = _NC * _NS
_PPW = SEQ // _NW
_C = 16
_NQ = _PPW // _C
_T = _NQ * BATCH
_LANES = EMB_DIM // 16


def _sinusoidal_pe_np(max_len, d_model):
    pos = np.arange(max_len, dtype=np.float32)[:, None]
    div = np.exp(
        np.arange(0, d_model, 2, dtype=np.float32) * (-math.log(10000.0) / d_model)
    )
    pe = np.zeros((max_len, d_model), dtype=np.float32)
    pe[:, 0::2] = np.sin(pos * div)
    pe[:, 1::2] = np.cos(pos * div)
    return pe


_PE = _sinusoidal_pe_np(SEQ, EMB_DIM)


def _pack_pe_words(pe):
    u = pe.astype(np.float32).view(np.uint32)
    lsb = (u >> 16) & 1
    bf = ((u + 0x7FFF + lsb) >> 16).astype(np.uint32)
    blk = bf.reshape(SEQ, EMB_DIM // 32, 32)
    words = blk[:, :, :16] | (blk[:, :, 16:] << 16)
    return words.reshape(SEQ, EMB_DIM // 2).view(np.int32)


_PE_WORDS = _pack_pe_words(_PE)


def _body(x_hbm, table_hbm, pe_hbm, out_hbm,
          idx_v, pe0, pe1, rows0, rows1, rows2, rows3, outv0, outv1,
          gs0, gs1, gs2, gs3, ps0, ps1, ss0, ss1):
    wid = lax.axis_index("s") * _NC + lax.axis_index("c")
    pbase = wid * _PPW
    pe_bufs = (pe0, pe1)
    rows_bufs = (rows0, rows1, rows2, rows3)
    out_bufs = (outv0, outv1)
    gsem = (gs0, gs1, gs2, gs3)
    psem = (ps0, ps1)
    ssem = (ss0, ss1)

    for b in range(BATCH):
        pltpu.sync_copy(x_hbm.at[b, pl.ds(pbase, _PPW)], idx_v.at[b])

    def g_args(q, b, slot):
        return (table_hbm.at[idx_v.at[b, pl.ds(q * _C, _C)]],
                rows_bufs[slot], gsem[slot])

    def p_args(q, slot):
        return (pe_hbm.at[pl.ds(pbase + q * _C, _C)], pe_bufs[slot], psem[slot])

    def s_args(q, b, slot):
        tok = b * SEQ + pbase + q * _C
        return (out_bufs[slot], out_hbm.at[pl.ds(tok, _C)], ssem[slot])

    def fuse(rslot, pslot, oslot):
        rows, pe_b, out_b = rows_bufs[rslot], pe_bufs[pslot], out_bufs[oslot]
        hi_mask = jnp.full((16,), -65536, jnp.int32)
        sh16 = jnp.full((16,), 16, jnp.int32)

        def row(i, carry):
            for g in range(EMB_DIM // 32):
                w = pe_b[i, pl.ds(g * 16, 16)]
                pe_lo = lax.bitcast_convert_type(
                    lax.shift_left(w, sh16), jnp.float32)
                pe_hi = lax.bitcast_convert_type(
                    lax.bitwise_and(w, hi_mask), jnp.float32)
                out_b[i, pl.ds(g * 32, 16)] = (
                    rows[i, pl.ds(g * 32, 16)] * SCALE + pe_lo
                )
                out_b[i, pl.ds(g * 32 + 16, 16)] = (
                    rows[i, pl.ds(g * 32 + 16, 16)] * SCALE + pe_hi
                )
            return carry

        lax.fori_loop(0, _C, row, 0)

    for b in range(BATCH):
        pltpu.async_copy(*g_args(0, b, b))
    pltpu.async_copy(*p_args(0, 0))

    def outer(i, carry):
        for k in range(8):
            qq = k // 4
            b = k % 4
            q = 2 * i + qq
            oslot = k % 2
            if b == 0:
                def issue_pe():
                    pltpu.async_copy(*p_args(q + 1, 1 - qq))
                if qq == 0:
                    issue_pe()
                else:
                    pl.when(i <= _NQ // 2 - 2)(issue_pe)
                pltpu.make_async_copy(*p_args(q, qq)).wait()
            pltpu.make_async_copy(*g_args(q, b, b)).wait()
            def wait_store():
                pltpu.make_async_copy(*s_args(q, b, oslot)).wait()
            if k < 2:
                pl.when(i >= 1)(wait_store)
            else:
                wait_store()
            fuse(b, qq, oslot)
            pltpu.async_copy(*s_args(q, b, oslot))
            def issue_gather():
                pltpu.async_copy(*g_args(q + 1, b, b))
            if qq == 1:
                pl.when(i <= _NQ // 2 - 2)(issue_gather)
            else:
                issue_gather()
        return carry

    lax.fori_loop(0, _NQ // 2, outer, 0)

    pltpu.make_async_copy(*s_args(_NQ - 1, 2, 0)).wait()
    pltpu.make_async_copy(*s_args(_NQ - 1, 3, 1)).wait()


@jax.jit
def kernel(x, token_table):
    x32 = x.astype(jnp.int32)
    pe = jnp.asarray(_PE_WORDS)
    mesh = plsc.VectorSubcoreMesh(core_axis_name="c", subcore_axis_name="s")
    run = functools.partial(
        pl.kernel,
        mesh=mesh,
        out_type=jax.ShapeDtypeStruct((BATCH * SEQ, EMB_DIM), jnp.float32),
        scratch_types=(
            [pltpu.VMEM((BATCH, _PPW), jnp.int32)]
            + [pltpu.VMEM((_C, EMB_DIM // 2), jnp.int32) for _ in range(2)]
            + [pltpu.VMEM((_C, EMB_DIM), jnp.float32) for _ in range(6)]
            + [pltpu.SemaphoreType.DMA for _ in range(8)]
        ),
    )(_body)
    out = run(x32, token_table, pe)
    return out.reshape(BATCH, SEQ, EMB_DIM)


# speedup vs baseline: 1.4556x; 1.4556x over previous
import functools
import math

import jax
import jax.numpy as jnp
import numpy as np
from jax import lax
from jax.experimental import pallas as pl
from jax.experimental.pallas import tpu as pltpu
from jax.experimental.pallas import tpu_sc as plsc

VOCAB = 100000
EMB_DIM = 1024
BATCH = 4
SEQ = 4096
SCALE = math.sqrt(float(EMB_DIM))

_NC = 2
_NS = 16
_NW = _NC * _NS
_PPW = SEQ // _NW
_C = 8
_NQ = _PPW // _C
_T = _NQ * BATCH
_LANES = EMB_DIM // 16


def _sinusoidal_pe_np(max_len, d_model):
    pos = np.arange(max_len, dtype=np.float32)[:, None]
    div = np.exp(
        np.arange(0, d_model, 2, dtype=np.float32) * (-math.log(10000.0) / d_model)
    )
    pe = np.zeros((max_len, d_model), dtype=np.float32)
    pe[:, 0::2] = np.sin(pos * div)
    pe[:, 1::2] = np.cos(pos * div)
    return pe


_PE = _sinusoidal_pe_np(SEQ, EMB_DIM)


def _pack_pe_words(pe):
    u = pe.astype(np.float32).view(np.uint32)
    lsb = (u >> 16) & 1
    bf = ((u + 0x7FFF + lsb) >> 16).astype(np.uint32)
    blk = bf.reshape(SEQ, EMB_DIM // 32, 32)
    words = blk[:, :, :16] | (blk[:, :, 16:] << 16)
    return words.reshape(SEQ, EMB_DIM // 2).view(np.int32)


_PE_WORDS = _pack_pe_words(_PE)


def _body(x_hbm, table_hbm, pe_hbm, out_hbm,
          idx_v, pe0, pe1, rows0, rows1, rows2, rows3, outv0, outv1,
          gs0, gs1, gs2, gs3, ps0, ps1, ss0, ss1):
    wid = lax.axis_index("s") * _NC + lax.axis_index("c")
    pbase = wid * _PPW
    pe_bufs = (pe0, pe1)
    rows_bufs = (rows0, rows1, rows2, rows3)
    out_bufs = (outv0, outv1)
    gsem = (gs0, gs1, gs2, gs3)
    psem = (ps0, ps1)
    ssem = (ss0, ss1)

    pltpu.sync_copy(x_hbm.at[:, pl.ds(pbase, _PPW)], idx_v)

    def g_args(q, b, slot):
        return (table_hbm.at[idx_v.at[b, pl.ds(q * _C, _C)]],
                rows_bufs[slot], gsem[slot])

    def p_args(q, slot):
        return (pe_hbm.at[pl.ds(pbase + q * _C, _C)], pe_bufs[slot], psem[slot])

    def s_args(q, b, slot):
        tok = b * SEQ + pbase + q * _C
        return (out_bufs[slot], out_hbm.at[pl.ds(tok, _C)], ssem[slot])

    def fuse(rslot, pslot, oslot):
        rows, pe_b, out_b = rows_bufs[rslot], pe_bufs[pslot], out_bufs[oslot]
        hi_mask = jnp.full((16,), -65536, jnp.int32)
        sh16 = jnp.full((16,), 16, jnp.int32)

        def row(i, carry):
            for g in range(EMB_DIM // 32):
                w = pe_b[i, pl.ds(g * 16, 16)]
                pe_lo = lax.bitcast_convert_type(
                    lax.shift_left(w, sh16), jnp.float32)
                pe_hi = lax.bitcast_convert_type(
                    lax.bitwise_and(w, hi_mask), jnp.float32)
                out_b[i, pl.ds(g * 32, 16)] = (
                    rows[i, pl.ds(g * 32, 16)] * SCALE + pe_lo
                )
                out_b[i, pl.ds(g * 32 + 16, 16)] = (
                    rows[i, pl.ds(g * 32 + 16, 16)] * SCALE + pe_hi
                )
            return carry

        lax.fori_loop(0, _C, row, 0)

    for b in range(BATCH):
        pltpu.async_copy(*g_args(0, b, b))
    pltpu.async_copy(*p_args(0, 0))

    def outer(i, carry):
        for k in range(8):
            qq = k // 4
            b = k % 4
            q = 2 * i + qq
            oslot = k % 2
            if b == 0:
                def issue_pe():
                    pltpu.async_copy(*p_args(q + 1, 1 - qq))
                if qq == 0:
                    issue_pe()
                else:
                    pl.when(i <= _NQ // 2 - 2)(issue_pe)
                pltpu.make_async_copy(*p_args(q, qq)).wait()
            pltpu.make_async_copy(*g_args(q, b, b)).wait()
            def wait_store():
                pltpu.make_async_copy(*s_args(q, b, oslot)).wait()
            if k < 2:
                pl.when(i >= 1)(wait_store)
            else:
                wait_store()
            fuse(b, qq, oslot)
            pltpu.async_copy(*s_args(q, b, oslot))
            def issue_gather():
                pltpu.async_copy(*g_args(q + 1, b, b))
            if qq == 1:
                pl.when(i <= _NQ // 2 - 2)(issue_gather)
            else:
                issue_gather()
        return carry

    lax.fori_loop(0, _NQ // 2, outer, 0)

    pltpu.make_async_copy(*s_args(_NQ - 1, 2, 0)).wait()
    pltpu.make_async_copy(*s_args(_NQ - 1, 3, 1)).wait()


@jax.jit
def kernel(x, token_table):
    x32 = x.astype(jnp.int32)
    pe = jnp.asarray(_PE_WORDS)
    mesh = plsc.VectorSubcoreMesh(core_axis_name="c", subcore_axis_name="s")
    run = functools.partial(
        pl.kernel,
        mesh=mesh,
        out_type=jax.ShapeDtypeStruct((BATCH * SEQ, EMB_DIM), jnp.float32),
        scratch_types=(
            [pltpu.VMEM((BATCH, _PPW), jnp.int32)]
            + [pltpu.VMEM((_C, EMB_DIM // 2), jnp.int32) for _ in range(2)]
            + [pltpu.VMEM((_C, EMB_DIM), jnp.float32) for _ in range(6)]
            + [pltpu.SemaphoreType.DMA for _ in range(8)]
        ),
    )(_body)
    out = run(x32, token_table, pe)
    return out.reshape(BATCH, SEQ, EMB_DIM)


# speedup vs baseline: 1.4659x; 1.0071x over previous
import functools
import math

import jax
import jax.numpy as jnp
import numpy as np
from jax import lax
from jax.experimental import pallas as pl
from jax.experimental.pallas import tpu as pltpu
from jax.experimental.pallas import tpu_sc as plsc

VOCAB = 100000
EMB_DIM = 1024
BATCH = 4
SEQ = 4096
SCALE = math.sqrt(float(EMB_DIM))

_NC = 2
_NS = 16
_NW = _NC * _NS
_PPW = SEQ // _NW
_C = 8
_NQ = _PPW // _C


def _sinusoidal_pe_np(max_len, d_model):
    pos = np.arange(max_len, dtype=np.float32)[:, None]
    div = np.exp(
        np.arange(0, d_model, 2, dtype=np.float32) * (-math.log(10000.0) / d_model)
    )
    pe = np.zeros((max_len, d_model), dtype=np.float32)
    pe[:, 0::2] = np.sin(pos * div)
    pe[:, 1::2] = np.cos(pos * div)
    return pe


_PE = _sinusoidal_pe_np(SEQ, EMB_DIM)


def _pack_pe_words(pe):
    u = pe.astype(np.float32).view(np.uint32)
    lsb = (u >> 16) & 1
    bf = ((u + 0x7FFF + lsb) >> 16).astype(np.uint32)
    blk = bf.reshape(SEQ, EMB_DIM // 32, 32)
    words = blk[:, :, :16] | (blk[:, :, 16:] << 16)
    return words.reshape(SEQ, EMB_DIM // 2).view(np.int32)


_PE_WORDS = _pack_pe_words(_PE)


def _body(x_hbm, table_hbm, pe_hbm, out_hbm,
          idx_v, pe0, pe1, rows0, rows1, rows2, rows3, outv0, outv1,
          gs0, gs1, gs2, gs3, ps0, ps1, ss0, ss1):
    wid = lax.axis_index("s") * _NC + lax.axis_index("c")
    pbase = wid * _PPW
    pe_bufs = (pe0, pe1)
    rows_bufs = (rows0, rows1, rows2, rows3)
    out_bufs = (outv0, outv1)
    gsem = (gs0, gs1, gs2, gs3)
    psem = (ps0, ps1)
    ssem = (ss0, ss1)

    pltpu.sync_copy(x_hbm.at[:, pl.ds(pbase, _PPW)], idx_v)

    def g_args(q, b, slot):
        return (table_hbm.at[idx_v.at[b, pl.ds(q * _C, _C)]],
                rows_bufs[slot], gsem[slot])

    def p_args(q, slot):
        return (pe_hbm.at[pl.ds(pbase + q * _C, _C)], pe_bufs[slot], psem[slot])

    def s_args(q, b, slot):
        tok = b * SEQ + pbase + q * _C
        return (out_bufs[slot], out_hbm.at[pl.ds(tok, _C)], ssem[slot])

    def fuse(rslot, pslot, oslot):
        rows, pe_b, out_b = rows_bufs[rslot], pe_bufs[pslot], out_bufs[oslot]
        hi_mask = jnp.full((16,), -65536, jnp.int32)
        sh16 = jnp.full((16,), 16, jnp.int32)

        def row(i, carry):
            for g in range(EMB_DIM // 32):
                w = pe_b[i, pl.ds(g * 16, 16)]
                pe_lo = lax.bitcast_convert_type(
                    lax.shift_left(w, sh16), jnp.float32)
                pe_hi = lax.bitcast_convert_type(
                    lax.bitwise_and(w, hi_mask), jnp.float32)
                out_b[i, pl.ds(g * 32, 16)] = (
                    rows[i, pl.ds(g * 32, 16)] * SCALE + pe_lo
                )
                out_b[i, pl.ds(g * 32 + 16, 16)] = (
                    rows[i, pl.ds(g * 32 + 16, 16)] * SCALE + pe_hi
                )
            return carry

        lax.fori_loop(0, _C, row, 0)

    for b in range(BATCH):
        pltpu.async_copy(*g_args(0, b, b))
    pltpu.async_copy(*p_args(0, 0))

    def outer(i, carry):
        for k in range(8):
            qq = k // 4
            b = k % 4
            q = 2 * i + qq
            oslot = k % 2
            if b == 0:
                def issue_pe():
                    pltpu.async_copy(*p_args(q + 1, 1 - qq))
                if qq == 0:
                    issue_pe()
                else:
                    pl.when(i <= _NQ // 2 - 2)(issue_pe)
                pltpu.make_async_copy(*p_args(q, qq)).wait()
            pltpu.make_async_copy(*g_args(q, b, b)).wait()
            def wait_store():
                pltpu.make_async_copy(*s_args(q, b, oslot)).wait()
            if k < 2:
                pl.when(i >= 1)(wait_store)
            else:
                wait_store()
            fuse(b, qq, oslot)
            pltpu.async_copy(*s_args(q, b, oslot))
            def issue_gather():
                pltpu.async_copy(*g_args(q + 1, b, b))
            if qq == 1:
                pl.when(i <= _NQ // 2 - 2)(issue_gather)
            else:
                issue_gather()
        return carry

    lax.fori_loop(0, _NQ // 2, outer, 0)

    pltpu.make_async_copy(*s_args(_NQ - 1, 2, 0)).wait()
    pltpu.make_async_copy(*s_args(_NQ - 1, 3, 1)).wait()


@jax.jit
def kernel(x, token_table):
    x32 = x.astype(jnp.int32)
    pe = jnp.asarray(_PE_WORDS)
    mesh = plsc.VectorSubcoreMesh(core_axis_name="c", subcore_axis_name="s")
    run = functools.partial(
        pl.kernel,
        mesh=mesh,
        out_type=jax.ShapeDtypeStruct((BATCH * SEQ, EMB_DIM), jnp.float32),
        scratch_types=(
            [pltpu.VMEM((BATCH, _PPW), jnp.int32)]
            + [pltpu.VMEM((_C, EMB_DIM // 2), jnp.int32) for _ in range(2)]
            + [pltpu.VMEM((_C, EMB_DIM), jnp.float32) for _ in range(6)]
            + [pltpu.SemaphoreType.DMA for _ in range(8)]
        ),
    )(_body)
    out = run(x32, token_table, pe)
    return out.reshape(BATCH, SEQ, EMB_DIM)
